# baseline (speedup 1.0000x reference)
import functools

import jax
import jax.numpy as jnp
from jax import lax
from jax.experimental import pallas as pl
from jax.experimental.pallas import tpu as pltpu
from jax.experimental.pallas import tpu_sc as plsc

B = 16384
D = 96
DENSE_IN = 384

NC = 2
NS = 16
NW = NC * NS
B_PER_W = B // NW
CHUNK = 128
NCHUNK = B_PER_W // CHUNK


def _sc_rowdma_body(idx_id, emb_id, out_id, idx_v, rows_a, rows_b, sem,
                    sem_out):
    wid = lax.axis_index("s") * NC + lax.axis_index("c")
    base = wid * B_PER_W

    pltpu.sync_copy(idx_id.at[pl.ds(base, B_PER_W)], idx_v)

    HALF = B_PER_W // 2
    bufs = (rows_a, rows_b)

    def fire_half(h, rows_v):
        def fire(c, _):
            vec = idx_v[pl.ds(h * HALF + c * 16, 16)]
            for j in range(16):
                r = vec[j]
                pltpu.async_copy(emb_id.at[pl.ds(r, 1)],
                                 rows_v.at[pl.ds(c * 16 + j, 1)], sem)
            return 0

        lax.fori_loop(0, HALF // 16, fire, 0, unroll=2)

    for h in range(2):
        fire_half(h, bufs[h])
        pltpu.make_async_copy(emb_id.at[pl.ds(0, HALF)], bufs[h], sem).wait()
        pltpu.async_copy(bufs[h], out_id.at[pl.ds(base + h * HALF, HALF)],
                         sem_out)
    for h in range(2):
        pltpu.make_async_copy(bufs[h], out_id.at[pl.ds(base, HALF)],
                              sem_out).wait()


@jax.jit
def _sc_rowdma_gather(track_ids, emb_id):
    mesh = plsc.VectorSubcoreMesh(core_axis_name="c", subcore_axis_name="s")
    scratch = [
        pltpu.VMEM((B_PER_W,), jnp.int32),
        pltpu.VMEM((B_PER_W // 2, D), jnp.float32),
        pltpu.VMEM((B_PER_W // 2, D), jnp.float32),
        pltpu.SemaphoreType.DMA,
        pltpu.SemaphoreType.DMA,
    ]
    return pl.kernel(_sc_rowdma_body,
                     out_type=jax.ShapeDtypeStruct((B, D), jnp.float32),
                     mesh=mesh, scratch_types=scratch)(track_ids, emb_id)


def _sc_stream_body(idx_art, idx_tag, emb_art, emb_tag, out_art, out_tag,
                    idx_v, rows_a, rows_b, sem):
    wid = lax.axis_index("s") * NC + lax.axis_index("c")
    base = wid * B_PER_W

    pltpu.sync_copy(idx_art.at[pl.ds(base, B_PER_W)],
                    idx_v.at[pl.ds(0, B_PER_W)])
    pltpu.sync_copy(idx_tag.at[pl.ds(base, B_PER_W)],
                    idx_v.at[pl.ds(B_PER_W, B_PER_W)])

    for t, (tbl, buf) in enumerate(((emb_art, rows_a), (emb_tag, rows_b))):
        for c in range(NCHUNK):
            pltpu.async_copy(
                tbl.at[idx_v.at[pl.ds(t * B_PER_W + c * CHUNK, CHUNK)]],
                buf.at[pl.ds(c * CHUNK, CHUNK)], sem)
    pltpu.make_async_copy(emb_art.at[pl.ds(0, B_PER_W)], rows_a, sem).wait()
    pltpu.make_async_copy(emb_tag.at[pl.ds(0, B_PER_W)], rows_b, sem).wait()
    pltpu.sync_copy(rows_a, out_art.at[pl.ds(base, B_PER_W)])
    pltpu.sync_copy(rows_b, out_tag.at[pl.ds(base, B_PER_W)])


@jax.jit
def _sc_stream_gather(track_artists, track_tags, emb_art, emb_tag):
    mesh = plsc.VectorSubcoreMesh(core_axis_name="c", subcore_axis_name="s")
    out_type = (
        jax.ShapeDtypeStruct((B, D), jnp.float32),
        jax.ShapeDtypeStruct((B, D), jnp.float32),
    )
    scratch = [
        pltpu.VMEM((2 * B_PER_W,), jnp.int32),
        pltpu.VMEM((B_PER_W, D), jnp.float32),
        pltpu.VMEM((B_PER_W, D), jnp.float32),
        pltpu.SemaphoreType.DMA,
    ]
    return pl.kernel(
        _sc_stream_body, out_type=out_type, mesh=mesh, scratch_types=scratch,
        compiler_params=pltpu.CompilerParams(use_tc_tiling_on_sc=False))(
        track_artists, track_tags, emb_art, emb_tag)


def _gelu(x):
    return 0.5 * x * (1.0 + lax.erf(x * 0.7071067811865476))


def _ln(x, eps=1e-5):
    m = jnp.mean(x, axis=-1, keepdims=True)
    xc = x - m
    v = jnp.mean(xc * xc, axis=-1, keepdims=True)
    return xc * lax.rsqrt(v + eps)


def _mlp_body(e_id, e_art, e_tag, names, wd, bd, w1a, w1b, w1c, w1d, b1,
              w2, b2, w3, b3, out):
    d = _gelu(jnp.dot(names[...], wd[...],
                      preferred_element_type=jnp.float32) + bd[...])
    t = (jnp.dot(e_id[...], w1a[...], preferred_element_type=jnp.float32)
         + jnp.dot(e_art[...], w1b[...], preferred_element_type=jnp.float32)
         + jnp.dot(e_tag[...], w1c[...], preferred_element_type=jnp.float32)
         + jnp.dot(d, w1d[...], preferred_element_type=jnp.float32)
         + b1[...])
    h = _gelu(_ln(t))
    u = jnp.dot(h, w2[...], preferred_element_type=jnp.float32) + b2[...]
    h2 = _gelu(_ln(u))
    out[...] = _gelu(jnp.dot(h2, w3[...],
                             preferred_element_type=jnp.float32) + b3[...])


@functools.partial(jax.jit, static_argnames=("bs",))
def _mlp(e_id, e_art, e_tag, names, Wd, bd, W1, b1, W2, b2, W3, b3, bs=1024):
    grid = (B // bs,)
    out_dim = b3.shape[-1]

    def rows(i):
        return (i, 0)

    def whole(i):
        return (0, 0)

    w1a, w1b, w1c, w1d = W1[0:D], W1[D:2 * D], W1[2 * D:3 * D], W1[3 * D:]
    return pl.pallas_call(
        _mlp_body,
        grid=grid,
        in_specs=[
            pl.BlockSpec((bs, D), rows),
            pl.BlockSpec((bs, D), rows),
            pl.BlockSpec((bs, D), rows),
            pl.BlockSpec((bs, DENSE_IN), rows),
            pl.BlockSpec(Wd.shape, whole),
            pl.BlockSpec((1, bd.shape[-1]), whole),
            pl.BlockSpec(w1a.shape, whole),
            pl.BlockSpec(w1b.shape, whole),
            pl.BlockSpec(w1c.shape, whole),
            pl.BlockSpec(w1d.shape, whole),
            pl.BlockSpec((1, b1.shape[-1]), whole),
            pl.BlockSpec(W2.shape, whole),
            pl.BlockSpec((1, b2.shape[-1]), whole),
            pl.BlockSpec(W3.shape, whole),
            pl.BlockSpec((1, b3.shape[-1]), whole),
        ],
        out_specs=pl.BlockSpec((bs, out_dim), rows),
        out_shape=jax.ShapeDtypeStruct((B, out_dim), jnp.float32),
    )(e_id, e_art, e_tag, names, Wd, bd.reshape(1, -1), w1a, w1b, w1c, w1d,
      b1.reshape(1, -1), W2, b2.reshape(1, -1), W3, b3.reshape(1, -1))


def kernel(track_ids, track_artists, track_tags, track_names, emb_id, emb_art,
           emb_tag, Wd, bd, W1, b1, W2, b2, W3, b3):
    e_id = _sc_rowdma_gather(track_ids, emb_id)
    e_art, e_tag = _sc_stream_gather(track_artists, track_tags, emb_art,
                                     emb_tag)
    return _mlp(e_id, e_art, e_tag, track_names, Wd, bd, W1, b1, W2, b2,
                W3, b3)


# speedup vs baseline: 1.3400x; 1.3400x over previous
import functools

import jax
import jax.numpy as jnp
from jax import lax
from jax.experimental import pallas as pl
from jax.experimental.pallas import tpu as pltpu
from jax.experimental.pallas import tpu_sc as plsc

B = 16384
D = 96
DENSE_IN = 384

NC = 2
NS = 16
NW = NC * NS
B_PER_W = B // NW
CHUNK = 128
NCHUNK = B_PER_W // CHUNK


def _sc_rowdma_body(idx_id, emb_id, out_id, idx_v, rows_a, rows_b, sem,
                    sem_out):
    wid = lax.axis_index("s") * NC + lax.axis_index("c")
    base = wid * B_PER_W

    pltpu.sync_copy(idx_id.at[pl.ds(base, B_PER_W)], idx_v)

    HALF = B_PER_W // 2
    bufs = (rows_a, rows_b)

    def fire_half(h, rows_v):
        def fire(c, _):
            vec = idx_v[pl.ds(h * HALF + c * 16, 16)]
            for j in range(16):
                r = vec[j]
                pltpu.async_copy(emb_id.at[pl.ds(r, 1)],
                                 rows_v.at[pl.ds(c * 16 + j, 1)], sem)
            return 0

        lax.fori_loop(0, HALF // 16, fire, 0, unroll=2)

    for h in range(2):
        fire_half(h, bufs[h])
        pltpu.make_async_copy(emb_id.at[pl.ds(0, HALF)], bufs[h], sem).wait()
        pltpu.async_copy(bufs[h], out_id.at[pl.ds(base + h * HALF, HALF)],
                         sem_out)
    for h in range(2):
        pltpu.make_async_copy(bufs[h], out_id.at[pl.ds(base, HALF)],
                              sem_out).wait()


@jax.jit
def _sc_rowdma_gather(track_ids, emb_id):
    mesh = plsc.VectorSubcoreMesh(core_axis_name="c", subcore_axis_name="s")
    scratch = [
        pltpu.VMEM((B_PER_W,), jnp.int32),
        pltpu.VMEM((B_PER_W // 2, D), jnp.float32),
        pltpu.VMEM((B_PER_W // 2, D), jnp.float32),
        pltpu.SemaphoreType.DMA,
        pltpu.SemaphoreType.DMA,
    ]
    return pl.kernel(_sc_rowdma_body,
                     out_type=jax.ShapeDtypeStruct((B, D), jnp.float32),
                     mesh=mesh, scratch_types=scratch)(track_ids, emb_id)


def _sc_stream_body(idx_art, idx_tag, emb_art, emb_tag, out_art, out_tag,
                    idx_v, rows_a, rows_b, sem):
    wid = lax.axis_index("s") * NC + lax.axis_index("c")
    base = wid * B_PER_W

    pltpu.sync_copy(idx_art.at[pl.ds(base, B_PER_W)],
                    idx_v.at[pl.ds(0, B_PER_W)])
    pltpu.sync_copy(idx_tag.at[pl.ds(base, B_PER_W)],
                    idx_v.at[pl.ds(B_PER_W, B_PER_W)])

    for t, (tbl, buf) in enumerate(((emb_art, rows_a), (emb_tag, rows_b))):
        for c in range(NCHUNK):
            pltpu.async_copy(
                tbl.at[idx_v.at[pl.ds(t * B_PER_W + c * CHUNK, CHUNK)]],
                buf.at[pl.ds(c * CHUNK, CHUNK)], sem)
    pltpu.make_async_copy(emb_art.at[pl.ds(0, B_PER_W)], rows_a, sem).wait()
    pltpu.make_async_copy(emb_tag.at[pl.ds(0, B_PER_W)], rows_b, sem).wait()
    pltpu.sync_copy(rows_a, out_art.at[pl.ds(base, B_PER_W)])
    pltpu.sync_copy(rows_b, out_tag.at[pl.ds(base, B_PER_W)])


@jax.jit
def _sc_stream_gather(track_artists, track_tags, emb_art, emb_tag):
    mesh = plsc.VectorSubcoreMesh(core_axis_name="c", subcore_axis_name="s")
    out_type = (
        jax.ShapeDtypeStruct((B, D), jnp.float32),
        jax.ShapeDtypeStruct((B, D), jnp.float32),
    )
    scratch = [
        pltpu.VMEM((2 * B_PER_W,), jnp.int32),
        pltpu.VMEM((B_PER_W, D), jnp.float32),
        pltpu.VMEM((B_PER_W, D), jnp.float32),
        pltpu.SemaphoreType.DMA,
    ]
    return pl.kernel(
        _sc_stream_body, out_type=out_type, mesh=mesh, scratch_types=scratch,
        compiler_params=pltpu.CompilerParams(use_tc_tiling_on_sc=False))(
        track_artists, track_tags, emb_art, emb_tag)


def _gelu(x):
    return 0.5 * x * (1.0 + lax.erf(x * 0.7071067811865476))


def _ln(x, eps=1e-5):
    m = jnp.mean(x, axis=-1, keepdims=True)
    xc = x - m
    v = jnp.mean(xc * xc, axis=-1, keepdims=True)
    return xc * lax.rsqrt(v + eps)


def _mlp_body(e_id, e_art, e_tag, names, wd, bd, w1a, w1b, w1c, w1d, b1,
              w2, b2, w3, b3, out):
    d = _gelu(jnp.dot(names[...], wd[...],
                      preferred_element_type=jnp.float32) + bd[...])
    t = (jnp.dot(e_id[...], w1a[...], preferred_element_type=jnp.float32)
         + jnp.dot(e_art[...], w1b[...], preferred_element_type=jnp.float32)
         + jnp.dot(e_tag[...], w1c[...], preferred_element_type=jnp.float32)
         + jnp.dot(d, w1d[...], preferred_element_type=jnp.float32)
         + b1[...])
    h = _gelu(_ln(t))
    u = jnp.dot(h, w2[...], preferred_element_type=jnp.float32) + b2[...]
    h2 = _gelu(_ln(u))
    out[...] = _gelu(jnp.dot(h2, w3[...],
                             preferred_element_type=jnp.float32) + b3[...])


@functools.partial(jax.jit, static_argnames=("bs",))
def _mlp(e_id, e_art, e_tag, names, Wd, bd, W1, b1, W2, b2, W3, b3, bs=1024):
    grid = (B // bs,)
    out_dim = b3.shape[-1]

    def rows(i):
        return (i, 0)

    def whole(i):
        return (0, 0)

    w1a, w1b, w1c, w1d = W1[0:D], W1[D:2 * D], W1[2 * D:3 * D], W1[3 * D:]
    return pl.pallas_call(
        _mlp_body,
        grid=grid,
        in_specs=[
            pl.BlockSpec((bs, D), rows),
            pl.BlockSpec((bs, D), rows),
            pl.BlockSpec((bs, D), rows),
            pl.BlockSpec((bs, DENSE_IN), rows),
            pl.BlockSpec(Wd.shape, whole),
            pl.BlockSpec((1, bd.shape[-1]), whole),
            pl.BlockSpec(w1a.shape, whole),
            pl.BlockSpec(w1b.shape, whole),
            pl.BlockSpec(w1c.shape, whole),
            pl.BlockSpec(w1d.shape, whole),
            pl.BlockSpec((1, b1.shape[-1]), whole),
            pl.BlockSpec(W2.shape, whole),
            pl.BlockSpec((1, b2.shape[-1]), whole),
            pl.BlockSpec(W3.shape, whole),
            pl.BlockSpec((1, b3.shape[-1]), whole),
        ],
        out_specs=pl.BlockSpec((bs, out_dim), rows),
        out_shape=jax.ShapeDtypeStruct((B, out_dim), jnp.float32),
    )(e_id, e_art, e_tag, names, Wd, bd.reshape(1, -1), w1a, w1b, w1c, w1d,
      b1.reshape(1, -1), W2, b2.reshape(1, -1), W3, b3.reshape(1, -1))


def kernel(track_ids, track_artists, track_tags, track_names, emb_id, emb_art,
           emb_tag, Wd, bd, W1, b1, W2, b2, W3, b3):
    e_id = _sc_rowdma_gather(track_ids, emb_id)
    e_art = _sc_rowdma_gather(track_artists, emb_art)
    e_tag = _sc_rowdma_gather(track_tags, emb_tag)
    return _mlp(e_id, e_art, e_tag, track_names, Wd, bd, W1, b1, W2, b2,
                W3, b3)


# speedup vs baseline: 1.3403x; 1.0002x over previous
import functools

import jax
import jax.numpy as jnp
from jax import lax
from jax.experimental import pallas as pl
from jax.experimental.pallas import tpu as pltpu
from jax.experimental.pallas import tpu_sc as plsc

B = 16384
D = 96
DENSE_IN = 384

NC = 2
NS = 16
NW = NC * NS
B_PER_W = B // NW


def _sc_rowdma_body(idx_id, emb_id, out_id, idx_v, rows_a, rows_b, sem,
                    sem_out):
    wid = lax.axis_index("s") * NC + lax.axis_index("c")
    base = wid * B_PER_W

    pltpu.sync_copy(idx_id.at[pl.ds(base, B_PER_W)], idx_v)

    HALF = B_PER_W // 2
    bufs = (rows_a, rows_b)

    def fire_half(h, rows_v):
        def fire(c, _):
            vec = idx_v[pl.ds(h * HALF + c * 16, 16)]
            for j in range(16):
                r = vec[j]
                pltpu.async_copy(emb_id.at[pl.ds(r, 1)],
                                 rows_v.at[pl.ds(c * 16 + j, 1)], sem)
            return 0

        lax.fori_loop(0, HALF // 16, fire, 0, unroll=2)

    for h in range(2):
        fire_half(h, bufs[h])
        pltpu.make_async_copy(emb_id.at[pl.ds(0, HALF)], bufs[h], sem).wait()
        pltpu.async_copy(bufs[h], out_id.at[pl.ds(base + h * HALF, HALF)],
                         sem_out)
    for h in range(2):
        pltpu.make_async_copy(bufs[h], out_id.at[pl.ds(base, HALF)],
                              sem_out).wait()


@jax.jit
def _sc_rowdma_gather(track_ids, emb_id):
    mesh = plsc.VectorSubcoreMesh(core_axis_name="c", subcore_axis_name="s")
    scratch = [
        pltpu.VMEM((B_PER_W,), jnp.int32),
        pltpu.VMEM((B_PER_W // 2, D), jnp.float32),
        pltpu.VMEM((B_PER_W // 2, D), jnp.float32),
        pltpu.SemaphoreType.DMA,
        pltpu.SemaphoreType.DMA,
    ]
    return pl.kernel(_sc_rowdma_body,
                     out_type=jax.ShapeDtypeStruct((B, D), jnp.float32),
                     mesh=mesh, scratch_types=scratch)(track_ids, emb_id)


def _gelu(x):
    return 0.5 * x * (1.0 + lax.erf(x * 0.7071067811865476))


def _ln(x, eps=1e-5):
    m = jnp.mean(x, axis=-1, keepdims=True)
    xc = x - m
    v = jnp.mean(xc * xc, axis=-1, keepdims=True)
    return xc * lax.rsqrt(v + eps)


def _mlp_body(e_id, e_art, e_tag, names, wd, bd, w1a, w1b, w1c, w1d, b1,
              w2, b2, w3, b3, out):
    d = _gelu(jnp.dot(names[...], wd[...],
                      preferred_element_type=jnp.float32) + bd[...])
    t = (jnp.dot(e_id[...], w1a[...], preferred_element_type=jnp.float32)
         + jnp.dot(e_art[...], w1b[...], preferred_element_type=jnp.float32)
         + jnp.dot(e_tag[...], w1c[...], preferred_element_type=jnp.float32)
         + jnp.dot(d, w1d[...], preferred_element_type=jnp.float32)
         + b1[...])
    h = _gelu(_ln(t))
    u = jnp.dot(h, w2[...], preferred_element_type=jnp.float32) + b2[...]
    h2 = _gelu(_ln(u))
    out[...] = _gelu(jnp.dot(h2, w3[...],
                             preferred_element_type=jnp.float32) + b3[...])


@functools.partial(jax.jit, static_argnames=("bs",))
def _mlp(e_id, e_art, e_tag, names, Wd, bd, W1, b1, W2, b2, W3, b3, bs=1024):
    grid = (B // bs,)
    out_dim = b3.shape[-1]

    def rows(i):
        return (i, 0)

    def whole(i):
        return (0, 0)

    w1a, w1b, w1c, w1d = W1[0:D], W1[D:2 * D], W1[2 * D:3 * D], W1[3 * D:]
    return pl.pallas_call(
        _mlp_body,
        grid=grid,
        in_specs=[
            pl.BlockSpec((bs, D), rows),
            pl.BlockSpec((bs, D), rows),
            pl.BlockSpec((bs, D), rows),
            pl.BlockSpec((bs, DENSE_IN), rows),
            pl.BlockSpec(Wd.shape, whole),
            pl.BlockSpec((1, bd.shape[-1]), whole),
            pl.BlockSpec(w1a.shape, whole),
            pl.BlockSpec(w1b.shape, whole),
            pl.BlockSpec(w1c.shape, whole),
            pl.BlockSpec(w1d.shape, whole),
            pl.BlockSpec((1, b1.shape[-1]), whole),
            pl.BlockSpec(W2.shape, whole),
            pl.BlockSpec((1, b2.shape[-1]), whole),
            pl.BlockSpec(W3.shape, whole),
            pl.BlockSpec((1, b3.shape[-1]), whole),
        ],
        out_specs=pl.BlockSpec((bs, out_dim), rows),
        out_shape=jax.ShapeDtypeStruct((B, out_dim), jnp.float32),
    )(e_id, e_art, e_tag, names, Wd, bd.reshape(1, -1), w1a, w1b, w1c, w1d,
      b1.reshape(1, -1), W2, b2.reshape(1, -1), W3, b3.reshape(1, -1))


def kernel(track_ids, track_artists, track_tags, track_names, emb_id, emb_art,
           emb_tag, Wd, bd, W1, b1, W2, b2, W3, b3):
    e_id = _sc_rowdma_gather(track_ids, emb_id)
    e_art = _sc_rowdma_gather(track_artists, emb_art)
    e_tag = _sc_rowdma_gather(track_tags, emb_tag)
    return _mlp(e_id, e_art, e_tag, track_names, Wd, bd, W1, b1, W2, b2,
                W3, b3)
